# baseline (speedup 1.0000x reference)
import functools

import jax
import jax.numpy as jnp
from jax import lax
from jax.experimental import pallas as pl
from jax.experimental.pallas import tpu as pltpu
from jax.experimental.pallas import tpu_sc as plsc

DIM = 32
NB = 8
TBLK = 128


def _make_transpose(V: int):
    info = plsc.get_sparse_core_info()
    NC, NS, L = info.num_cores, info.num_subcores, info.num_lanes
    NW = NC * NS
    n_full = V // TBLK
    tail = V - n_full * TBLK
    assert tail % 4 == 0
    niter = -(-(n_full) // NW)
    niter += (-niter) % 4

    mesh = plsc.VectorSubcoreMesh(core_axis_name="c", subcore_axis_name="s")

    @functools.partial(
        pl.kernel,
        out_type=jax.ShapeDtypeStruct((V // 4, 4 * DIM), jnp.float32),
        mesh=mesh,
        scratch_types=(
            [pltpu.VMEM((DIM, TBLK), jnp.float32) for _ in range(4)]
            + [pltpu.VMEM((TBLK // 4, 4 * DIM), jnp.float32) for _ in range(4)]
            + [pltpu.SemaphoreType.DMA for _ in range(8)]
        ),
        compiler_params=pltpu.CompilerParams(needs_layout_passes=False),
    )
    def tr(tt_hbm, tail_hbm, out_hbm, *rest):
        bufs = rest[:4]
        obufs = rest[4:8]
        gsems = rest[8:12]
        wsems = rest[12:16]
        wid = lax.axis_index("s") * NC + lax.axis_index("c")
        iota = lax.iota(jnp.int32, 16)

        def blk_of(i):
            return i * NW + wid

        def fire(blk, p):
            pltpu.async_copy(
                tt_hbm.at[:, pl.ds(blk * TBLK, TBLK)], bufs[p], gsems[p])

        def gdrain(p):
            pltpu.make_async_copy(
                tt_hbm.at[:, pl.ds(0, TBLK)], bufs[p], gsems[p]).wait()

        def wfire(blk, p):
            pltpu.async_copy(
                obufs[p], out_hbm.at[pl.ds(blk * (TBLK // 4), TBLK // 4)],
                wsems[p])

        def wdrain(p):
            pltpu.make_async_copy(
                obufs[p], out_hbm.at[pl.ds(0, TBLK // 4)], wsems[p]).wait()

        def transpose(p, nrow):
            buf, obuf = bufs[p], obufs[p]
            for r in range(nrow):
                vals = [
                    plsc.load_gather(
                        buf,
                        [iota + k * 16, jnp.full((16,), 4 * r + q, jnp.int32)])
                    for q in range(4) for k in range(2)
                ]
                i = 0
                for q in range(4):
                    for k in range(2):
                        obuf[r, pl.ds(q * 32 + k * 16, 16)] = vals[i]
                        i += 1

        for p in range(4):
            @pl.when(blk_of(p) < n_full)
            def _():
                fire(blk_of(p), p)

        def body(g, carry):
            for p in range(4):
                i = 4 * g + p
                blk = blk_of(i)

                @pl.when(blk < n_full)
                def _():
                    @pl.when(g > 0)
                    def _():
                        wdrain(p)

                    gdrain(p)
                    transpose(p, TBLK // 4)
                    wfire(blk, p)

                @pl.when(blk_of(i + 4) < n_full)
                def _():
                    fire(blk_of(i + 4), p)
            return carry

        lax.fori_loop(0, niter // 4, body, 0)
        for p in range(4):
            wdrain(p)

        if tail:
            @pl.when(wid == 0)
            def _():
                pltpu.sync_copy(tail_hbm, obufs[0].at[pl.ds(0, tail // 4)])
                pltpu.sync_copy(
                    obufs[0].at[pl.ds(0, tail // 4)],
                    out_hbm.at[pl.ds(n_full * (TBLK // 4), tail // 4)])

    return tr


def _make_gather(BATCH: int, HIST: int, V: int):
    info = plsc.get_sparse_core_info()
    NC, NS, L = info.num_cores, info.num_subcores, info.num_lanes
    NW = NC * NS
    assert BATCH % (NW * L) == 0 and HIST % NB == 0
    bw = BATCH // NW
    toks_w = bw * HIST
    nblk = bw // L
    n_grp = HIST // NB

    mesh = plsc.VectorSubcoreMesh(core_axis_name="c", subcore_axis_name="s")

    @functools.partial(
        pl.kernel,
        out_type=jax.ShapeDtypeStruct((HIST, DIM, BATCH), jnp.float32),
        mesh=mesh,
        scratch_types=(
            [pltpu.VMEM((toks_w,), jnp.int32),
             pltpu.VMEM((HIST, bw), jnp.int32)]
            + [pltpu.VMEM((bw, DIM), jnp.float32) for _ in range(NB)]
            + [pltpu.VMEM((DIM, bw), jnp.float32) for _ in range(NB)]
            + [pltpu.SemaphoreType.DMA for _ in range(2 * NB)]
        ),
        compiler_params=pltpu.CompilerParams(
            use_tc_tiling_on_sc=False, needs_layout_passes=False),
    )
    def emb(idx_hbm, table_hbm, out_hbm, idx_v, tok_t, *rest):
        g = rest[:NB]
        ob = rest[NB:2 * NB]
        gsem = rest[2 * NB:3 * NB]
        wsem = rest[3 * NB:4 * NB]
        wid = lax.axis_index("s") * NC + lax.axis_index("c")
        pltpu.sync_copy(idx_hbm.at[pl.ds(wid * toks_w, toks_w)], idx_v)

        iota = lax.iota(jnp.int32, 16)
        iota_h = iota * HIST

        def transform(h, carry):
            vals = [
                plsc.load_gather(idx_v, [iota_h + (blk * 16 * HIST) + h])
                for blk in range(nblk)
            ]
            for blk in range(nblk):
                tok_t[h, pl.ds(blk * 16, 16)] = vals[blk]
            return carry

        lax.fori_loop(0, HIST, transform, 0)

        def fire(h, b):
            pltpu.async_copy(table_hbm.at[tok_t.at[h]], g[b], gsem[b])

        def gdrain(b):
            pltpu.make_async_copy(table_hbm.at[tok_t.at[0]], g[b],
                                  gsem[b]).wait()

        def wfire(h, b):
            pltpu.async_copy(ob[b], out_hbm.at[h, :, pl.ds(wid * bw, bw)],
                             wsem[b])

        def wdrain(b):
            pltpu.make_async_copy(ob[b],
                                  out_hbm.at[0, :, pl.ds(wid * bw, bw)],
                                  wsem[b]).wait()

        def extract(buf, b):
            for blk in range(nblk):
                rows = iota + blk * 16
                vals = [
                    plsc.load_gather(
                        buf, [rows, jnp.full((16,), d, jnp.int32)])
                    for d in range(DIM)
                ]
                for d in range(DIM):
                    ob[b][d, pl.ds(blk * 16, 16)] = vals[d]

        for b in range(NB):
            fire(b, b)

        def body(grp, carry):
            h0 = grp * NB
            for b in range(NB):
                @pl.when(grp > 0)
                def _():
                    wdrain(b)

                gdrain(b)
                extract(g[b], b)
                wfire(h0 + b, b)

                @pl.when(h0 + b + NB < HIST)
                def _():
                    fire(h0 + b + NB, b)
            return carry

        lax.fori_loop(0, n_grp, body, 0)
        for b in range(NB):
            wdrain(b)

    return emb


def kernel(token_ids, embed):
    BATCH, HIST = token_ids.shape
    V = embed.shape[0]
    idx = token_ids.reshape(-1).astype(jnp.int32)
    n_full = V // TBLK
    tail = V - n_full * TBLK
    tail_rows = embed[n_full * TBLK:].reshape(tail // 4, 4 * DIM)
    table4 = _make_transpose(V)(jnp.transpose(embed), tail_rows)
    table = table4.reshape(V, DIM)
    out = _make_gather(BATCH, HIST, V)(idx, table)
    return jnp.transpose(out, (2, 0, 1))


# speedup vs baseline: 1.0767x; 1.0767x over previous
import functools

import jax
import jax.numpy as jnp
from jax import lax
from jax.experimental import pallas as pl
from jax.experimental.pallas import tpu as pltpu
from jax.experimental.pallas import tpu_sc as plsc

DIM = 32
NB = 8
TBLK = 128


def _make_transpose(V: int):
    info = plsc.get_sparse_core_info()
    NC, NS, L = info.num_cores, info.num_subcores, info.num_lanes
    NW = NC * NS
    n_full = V // TBLK
    tail = V - n_full * TBLK
    assert tail % 4 == 0
    niter = -(-(n_full) // NW)
    niter += (-niter) % 4

    mesh = plsc.VectorSubcoreMesh(core_axis_name="c", subcore_axis_name="s")

    @functools.partial(
        pl.kernel,
        out_type=jax.ShapeDtypeStruct((V // 4, 4 * DIM), jnp.float32),
        mesh=mesh,
        scratch_types=(
            [pltpu.VMEM((DIM, TBLK + 1), jnp.float32) for _ in range(4)]
            + [pltpu.VMEM((TBLK // 4, 4 * DIM), jnp.float32) for _ in range(4)]
            + [pltpu.SemaphoreType.DMA for _ in range(8)]
        ),
        compiler_params=pltpu.CompilerParams(needs_layout_passes=False),
    )
    def tr(tt_hbm, tail_hbm, out_hbm, *rest):
        bufs = rest[:4]
        obufs = rest[4:8]
        gsems = rest[8:12]
        wsems = rest[12:16]
        wid = lax.axis_index("s") * NC + lax.axis_index("c")
        iota = lax.iota(jnp.int32, 16)

        def blk_of(i):
            return i * NW + wid

        def fire(blk, p):
            pltpu.async_copy(
                tt_hbm.at[:, pl.ds(blk * TBLK, TBLK)],
                bufs[p].at[:, pl.ds(0, TBLK)], gsems[p])

        def gdrain(p):
            pltpu.make_async_copy(
                tt_hbm.at[:, pl.ds(0, TBLK)],
                bufs[p].at[:, pl.ds(0, TBLK)], gsems[p]).wait()

        def wfire(blk, p):
            pltpu.async_copy(
                obufs[p], out_hbm.at[pl.ds(blk * (TBLK // 4), TBLK // 4)],
                wsems[p])

        def wdrain(p):
            pltpu.make_async_copy(
                obufs[p], out_hbm.at[pl.ds(0, TBLK // 4)], wsems[p]).wait()

        def transpose(p, nrow):
            buf, obuf = bufs[p], obufs[p]
            for r in range(nrow):
                vals = [
                    plsc.load_gather(
                        buf,
                        [iota + k * 16, jnp.full((16,), 4 * r + q, jnp.int32)])
                    for q in range(4) for k in range(2)
                ]
                i = 0
                for q in range(4):
                    for k in range(2):
                        obuf[r, pl.ds(q * 32 + k * 16, 16)] = vals[i]
                        i += 1

        for p in range(4):
            @pl.when(blk_of(p) < n_full)
            def _():
                fire(blk_of(p), p)

        def body(g, carry):
            for p in range(4):
                i = 4 * g + p
                blk = blk_of(i)

                @pl.when(blk < n_full)
                def _():
                    @pl.when(g > 0)
                    def _():
                        wdrain(p)

                    gdrain(p)
                    transpose(p, TBLK // 4)
                    wfire(blk, p)

                @pl.when(blk_of(i + 4) < n_full)
                def _():
                    fire(blk_of(i + 4), p)
            return carry

        lax.fori_loop(0, niter // 4, body, 0)
        for p in range(4):
            wdrain(p)

        if tail:
            @pl.when(wid == 0)
            def _():
                pltpu.sync_copy(tail_hbm, obufs[0].at[pl.ds(0, tail // 4)])
                pltpu.sync_copy(
                    obufs[0].at[pl.ds(0, tail // 4)],
                    out_hbm.at[pl.ds(n_full * (TBLK // 4), tail // 4)])

    return tr


def _make_gather(BATCH: int, HIST: int, V: int):
    info = plsc.get_sparse_core_info()
    NC, NS, L = info.num_cores, info.num_subcores, info.num_lanes
    NW = NC * NS
    assert BATCH % (NW * L) == 0 and HIST % NB == 0
    bw = BATCH // NW
    toks_w = bw * HIST
    nblk = bw // L
    n_grp = HIST // NB

    mesh = plsc.VectorSubcoreMesh(core_axis_name="c", subcore_axis_name="s")

    @functools.partial(
        pl.kernel,
        out_type=jax.ShapeDtypeStruct((HIST, DIM, BATCH), jnp.float32),
        mesh=mesh,
        scratch_types=(
            [pltpu.VMEM((toks_w,), jnp.int32),
             pltpu.VMEM((HIST, bw), jnp.int32)]
            + [pltpu.VMEM((bw, DIM), jnp.float32) for _ in range(NB)]
            + [pltpu.VMEM((DIM, bw + 1), jnp.float32) for _ in range(NB)]
            + [pltpu.SemaphoreType.DMA for _ in range(2 * NB)]
        ),
        compiler_params=pltpu.CompilerParams(
            use_tc_tiling_on_sc=False, needs_layout_passes=False),
    )
    def emb(idx_hbm, table_hbm, out_hbm, idx_v, tok_t, *rest):
        g = rest[:NB]
        ob = rest[NB:2 * NB]
        gsem = rest[2 * NB:3 * NB]
        wsem = rest[3 * NB:4 * NB]
        wid = lax.axis_index("s") * NC + lax.axis_index("c")
        pltpu.sync_copy(idx_hbm.at[pl.ds(wid * toks_w, toks_w)], idx_v)

        iota = lax.iota(jnp.int32, 16)
        iota_h = iota * HIST

        def transform(h, carry):
            vals = [
                plsc.load_gather(idx_v, [iota_h + (blk * 16 * HIST) + h])
                for blk in range(nblk)
            ]
            for blk in range(nblk):
                tok_t[h, pl.ds(blk * 16, 16)] = vals[blk]
            return carry

        lax.fori_loop(0, HIST, transform, 0)

        def fire(h, b):
            pltpu.async_copy(table_hbm.at[tok_t.at[h]], g[b], gsem[b])

        def gdrain(b):
            pltpu.make_async_copy(table_hbm.at[tok_t.at[0]], g[b],
                                  gsem[b]).wait()

        def wfire(h, b):
            pltpu.async_copy(ob[b].at[:, pl.ds(0, bw)],
                             out_hbm.at[h, :, pl.ds(wid * bw, bw)],
                             wsem[b])

        def wdrain(b):
            pltpu.make_async_copy(ob[b].at[:, pl.ds(0, bw)],
                                  out_hbm.at[0, :, pl.ds(wid * bw, bw)],
                                  wsem[b]).wait()

        def extract(buf, b):
            iota2 = iota + 16
            for r in range(bw):
                v0 = buf[r, pl.ds(0, 16)]
                v1 = buf[r, pl.ds(16, 16)]
                col = jnp.full((16,), r, jnp.int32)
                plsc.store_scatter(ob[b], [iota, col], v0)
                plsc.store_scatter(ob[b], [iota2, col], v1)

        for b in range(NB):
            fire(b, b)

        def body(grp, carry):
            h0 = grp * NB
            for b in range(NB):
                @pl.when(grp > 0)
                def _():
                    wdrain(b)

                gdrain(b)
                extract(g[b], b)
                wfire(h0 + b, b)

                @pl.when(h0 + b + NB < HIST)
                def _():
                    fire(h0 + b + NB, b)
            return carry

        lax.fori_loop(0, n_grp, body, 0)
        for b in range(NB):
            wdrain(b)

    return emb


def kernel(token_ids, embed):
    BATCH, HIST = token_ids.shape
    V = embed.shape[0]
    idx = token_ids.reshape(-1).astype(jnp.int32)
    n_full = V // TBLK
    tail = V - n_full * TBLK
    tail_rows = embed[n_full * TBLK:].reshape(tail // 4, 4 * DIM)
    table4 = _make_transpose(V)(jnp.transpose(embed), tail_rows)
    table = table4.reshape(V, DIM)
    out = _make_gather(BATCH, HIST, V)(idx, table)
    return jnp.transpose(out, (2, 0, 1))


# speedup vs baseline: 1.1609x; 1.0782x over previous
import functools

import jax
import jax.numpy as jnp
from jax import lax
from jax.experimental import pallas as pl
from jax.experimental.pallas import tpu as pltpu
from jax.experimental.pallas import tpu_sc as plsc

DIM = 32
NB = 8
TBLK = 128


def _make_transpose(V: int):
    info = plsc.get_sparse_core_info()
    NC, NS, L = info.num_cores, info.num_subcores, info.num_lanes
    NW = NC * NS
    n_full = V // TBLK
    tail = V - n_full * TBLK
    assert tail % 4 == 0
    niter = -(-(n_full) // NW)
    niter += (-niter) % 4

    mesh = plsc.VectorSubcoreMesh(core_axis_name="c", subcore_axis_name="s")

    @functools.partial(
        pl.kernel,
        out_type=jax.ShapeDtypeStruct((V // 4, 4 * DIM), jnp.float32),
        mesh=mesh,
        scratch_types=(
            [pltpu.VMEM((DIM, TBLK + 1), jnp.float32) for _ in range(4)]
            + [pltpu.VMEM((TBLK // 4, 4 * DIM), jnp.float32) for _ in range(4)]
            + [pltpu.SemaphoreType.DMA for _ in range(8)]
        ),
        compiler_params=pltpu.CompilerParams(needs_layout_passes=False),
    )
    def tr(tt_hbm, tail_hbm, out_hbm, *rest):
        bufs = rest[:4]
        obufs = rest[4:8]
        gsems = rest[8:12]
        wsems = rest[12:16]
        wid = lax.axis_index("s") * NC + lax.axis_index("c")
        iota = lax.iota(jnp.int32, 16)

        def blk_of(i):
            return i * NW + wid

        def fire(blk, p):
            pltpu.async_copy(
                tt_hbm.at[:, pl.ds(blk * TBLK, TBLK)],
                bufs[p].at[:, pl.ds(0, TBLK)], gsems[p])

        def gdrain(p):
            pltpu.make_async_copy(
                tt_hbm.at[:, pl.ds(0, TBLK)],
                bufs[p].at[:, pl.ds(0, TBLK)], gsems[p]).wait()

        def wfire(blk, p):
            pltpu.async_copy(
                obufs[p], out_hbm.at[pl.ds(blk * (TBLK // 4), TBLK // 4)],
                wsems[p])

        def wdrain(p):
            pltpu.make_async_copy(
                obufs[p], out_hbm.at[pl.ds(0, TBLK // 4)], wsems[p]).wait()

        def transpose(p, nrow):
            buf, obuf = bufs[p], obufs[p]
            for r in range(nrow):
                vals = [
                    plsc.load_gather(
                        buf,
                        [iota + k * 16, jnp.full((16,), 4 * r + q, jnp.int32)])
                    for q in range(4) for k in range(2)
                ]
                i = 0
                for q in range(4):
                    for k in range(2):
                        obuf[r, pl.ds(q * 32 + k * 16, 16)] = vals[i]
                        i += 1

        for p in range(4):
            @pl.when(blk_of(p) < n_full)
            def _():
                fire(blk_of(p), p)

        def body(g, carry):
            for p in range(4):
                i = 4 * g + p
                blk = blk_of(i)

                @pl.when(blk < n_full)
                def _():
                    @pl.when(g > 0)
                    def _():
                        wdrain(p)

                    gdrain(p)
                    transpose(p, TBLK // 4)
                    wfire(blk, p)

                @pl.when(blk_of(i + 4) < n_full)
                def _():
                    fire(blk_of(i + 4), p)
            return carry

        lax.fori_loop(0, niter // 4, body, 0)
        for p in range(4):
            wdrain(p)

        if tail:
            @pl.when(wid == 0)
            def _():
                pltpu.sync_copy(tail_hbm, obufs[0].at[pl.ds(0, tail // 4)])
                pltpu.sync_copy(
                    obufs[0].at[pl.ds(0, tail // 4)],
                    out_hbm.at[pl.ds(n_full * (TBLK // 4), tail // 4)])

    return tr


def _make_gather(BATCH: int, HIST: int, V: int):
    info = plsc.get_sparse_core_info()
    NC, NS, L = info.num_cores, info.num_subcores, info.num_lanes
    NW = NC * NS
    assert BATCH % (NW * L) == 0 and HIST % NB == 0
    bw = BATCH // NW
    toks_w = bw * HIST
    nblk = bw // L
    n_grp = HIST // NB

    mesh = plsc.VectorSubcoreMesh(core_axis_name="c", subcore_axis_name="s")

    @functools.partial(
        pl.kernel,
        out_type=jax.ShapeDtypeStruct((HIST, DIM, BATCH), jnp.float32),
        mesh=mesh,
        scratch_types=(
            [pltpu.VMEM((toks_w,), jnp.int32),
             pltpu.VMEM((HIST, bw), jnp.int32)]
            + [pltpu.VMEM((bw, DIM), jnp.float32) for _ in range(NB)]
            + [pltpu.VMEM((DIM, bw + 1), jnp.float32) for _ in range(NB)]
            + [pltpu.SemaphoreType.DMA for _ in range(2 * NB)]
        ),
        compiler_params=pltpu.CompilerParams(
            use_tc_tiling_on_sc=False, needs_layout_passes=False),
    )
    def emb(idx_hbm, table_hbm, out_hbm, idx_v, tok_t, *rest):
        g = rest[:NB]
        ob = rest[NB:2 * NB]
        gsem = rest[2 * NB:3 * NB]
        wsem = rest[3 * NB:4 * NB]
        wid = lax.axis_index("s") * NC + lax.axis_index("c")
        pltpu.sync_copy(idx_hbm.at[pl.ds(wid * toks_w, toks_w)], idx_v)

        iota = lax.iota(jnp.int32, 16)
        iota_h = iota * HIST

        def transform(h, carry):
            vals = [
                plsc.load_gather(idx_v, [iota_h + (blk * 16 * HIST) + h])
                for blk in range(nblk)
            ]
            for blk in range(nblk):
                tok_t[h, pl.ds(blk * 16, 16)] = vals[blk]
            return carry

        lax.fori_loop(0, HIST, transform, 0)

        def fire(h, b):
            pltpu.async_copy(table_hbm.at[tok_t.at[h]], g[b], gsem[b])

        def gdrain(b):
            pltpu.make_async_copy(table_hbm.at[tok_t.at[0]], g[b],
                                  gsem[b]).wait()

        def wfire(h, b):
            pltpu.async_copy(ob[b].at[:, pl.ds(0, bw)],
                             out_hbm.at[h, :, pl.ds(wid * bw, bw)],
                             wsem[b])

        def wdrain(b):
            pltpu.make_async_copy(ob[b].at[:, pl.ds(0, bw)],
                                  out_hbm.at[0, :, pl.ds(wid * bw, bw)],
                                  wsem[b]).wait()

        def extract(buf, b):
            iota2 = iota + 16
            for r in range(bw):
                v0 = buf[r, pl.ds(0, 16)]
                v1 = buf[r, pl.ds(16, 16)]
                col = jnp.full((16,), r, jnp.int32)
                plsc.store_scatter(ob[b], [iota, col], v0)
                plsc.store_scatter(ob[b], [iota2, col], v1)

        for b in range(NB):
            fire(b, b)

        def body(grp, carry):
            h0 = grp * NB
            for b in range(NB):
                @pl.when(grp > 0)
                def _():
                    wdrain(b)

                gdrain(b)
                extract(g[b], b)
                wfire(h0 + b, b)

                @pl.when(h0 + b + NB < HIST)
                def _():
                    fire(h0 + b + NB, b)
            return carry

        lax.fori_loop(0, n_grp, body, 0)
        for b in range(NB):
            wdrain(b)

    return emb


def kernel(token_ids, embed):
    BATCH, HIST = token_ids.shape
    V = embed.shape[0]
    idx = token_ids.reshape(-1).astype(jnp.int32)
    out = _make_gather(BATCH, HIST, V)(idx, embed)
    return jnp.transpose(out, (2, 0, 1))


# speedup vs baseline: 1.1632x; 1.0020x over previous
import functools

import jax
import jax.numpy as jnp
from jax import lax
from jax.experimental import pallas as pl
from jax.experimental.pallas import tpu as pltpu
from jax.experimental.pallas import tpu_sc as plsc

DIM = 32
NB = 8


def _make_gather(BATCH: int, HIST: int, V: int):
    info = plsc.get_sparse_core_info()
    NC, NS, L = info.num_cores, info.num_subcores, info.num_lanes
    NW = NC * NS
    assert BATCH % (NW * L) == 0 and HIST % NB == 0
    bw = BATCH // NW
    toks_w = bw * HIST
    nblk = bw // L
    n_grp = HIST // NB

    mesh = plsc.VectorSubcoreMesh(core_axis_name="c", subcore_axis_name="s")

    @functools.partial(
        pl.kernel,
        out_type=jax.ShapeDtypeStruct((HIST, DIM, BATCH), jnp.float32),
        mesh=mesh,
        scratch_types=(
            [pltpu.VMEM((toks_w,), jnp.int32),
             pltpu.VMEM((HIST, bw), jnp.int32)]
            + [pltpu.VMEM((bw, DIM), jnp.float32) for _ in range(NB)]
            + [pltpu.VMEM((DIM, bw + 1), jnp.float32) for _ in range(NB)]
            + [pltpu.SemaphoreType.DMA for _ in range(2 * NB)]
        ),
        compiler_params=pltpu.CompilerParams(
            use_tc_tiling_on_sc=False, needs_layout_passes=False),
    )
    def emb(idx_hbm, table_hbm, out_hbm, idx_v, tok_t, *rest):
        g = rest[:NB]
        ob = rest[NB:2 * NB]
        gsem = rest[2 * NB:3 * NB]
        wsem = rest[3 * NB:4 * NB]
        wid = lax.axis_index("s") * NC + lax.axis_index("c")
        pltpu.sync_copy(idx_hbm.at[pl.ds(wid * toks_w, toks_w)], idx_v)

        iota = lax.iota(jnp.int32, 16)
        iota_h = iota * HIST

        def transform(h, carry):
            vals = [
                plsc.load_gather(idx_v, [iota_h + (blk * 16 * HIST) + h])
                for blk in range(nblk)
            ]
            for blk in range(nblk):
                tok_t[h, pl.ds(blk * 16, 16)] = vals[blk]
            return carry

        lax.fori_loop(0, HIST, transform, 0)

        def fire(h, b):
            pltpu.async_copy(table_hbm.at[tok_t.at[h]], g[b], gsem[b])

        def gdrain(b):
            pltpu.make_async_copy(table_hbm.at[tok_t.at[0]], g[b],
                                  gsem[b]).wait()

        def wfire(h, b):
            pltpu.async_copy(ob[b].at[:, pl.ds(0, bw)],
                             out_hbm.at[h, :, pl.ds(wid * bw, bw)],
                             wsem[b])

        def wdrain(b):
            pltpu.make_async_copy(ob[b].at[:, pl.ds(0, bw)],
                                  out_hbm.at[0, :, pl.ds(wid * bw, bw)],
                                  wsem[b]).wait()

        def extract(buf, b):
            iota2 = iota + 16
            for r in range(bw):
                v0 = buf[r, pl.ds(0, 16)]
                v1 = buf[r, pl.ds(16, 16)]
                col = jnp.full((16,), r, jnp.int32)
                plsc.store_scatter(ob[b], [iota, col], v0)
                plsc.store_scatter(ob[b], [iota2, col], v1)

        for b in range(NB):
            fire(b, b)

        def body(grp, carry):
            h0 = grp * NB
            for b in range(NB):
                @pl.when(grp > 0)
                def _():
                    wdrain(b)

                gdrain(b)
                extract(g[b], b)
                wfire(h0 + b, b)

                @pl.when(h0 + b + NB < HIST)
                def _():
                    fire(h0 + b + NB, b)
            return carry

        lax.fori_loop(0, n_grp, body, 0)
        for b in range(NB):
            wdrain(b)

    return emb


def kernel(token_ids, embed):
    BATCH, HIST = token_ids.shape
    V = embed.shape[0]
    idx = token_ids.reshape(-1).astype(jnp.int32)
    out = _make_gather(BATCH, HIST, V)(idx, embed)
    return jnp.transpose(out, (2, 0, 1))
